# Initial kernel scaffold; baseline (speedup 1.0000x reference)
import jax
import jax.numpy as jnp
from jax.experimental import pallas as pl


def kernel(attributes, E0, E1, E2, E3, W, b):
    raise NotImplementedError("write your pallas kernel here")


# speedup vs baseline: 6.1895x; 6.1895x over previous
import functools

import jax
import jax.numpy as jnp
from jax import lax
from jax.experimental import pallas as pl
from jax.experimental.pallas import tpu as pltpu
from jax.experimental.pallas import tpu_sc as plsc

B = 16384
BARS = 8
BINS = 8
ADIM = 32
NEMB = 512
ROWS = B * BARS
NTT = BINS ** 4

NC = 2
NS = 16
NW = NC * NS
RPW = ROWS // NW
CHUNK = 64
NCHUNK = RPW // CHUNK


def _tables_body(e0, e1, e2, e3, w, b, tt_ref):
    h = pl.program_id(0)
    r = lax.broadcasted_iota(jnp.int32, (64, 8), 0)
    c = lax.broadcasted_iota(jnp.int32, (64, 8), 1)
    s_lo = (c == (r & 7)).astype(jnp.float32)
    s_hi = (c == (r >> 3)).astype(jnp.float32)
    wv = w[...]
    dot = functools.partial(jnp.dot, preferred_element_type=jnp.float32)
    t0 = dot(e0[...], wv[0:32])
    t1 = dot(e1[...], wv[32:64])
    t2 = dot(e2[...], wv[64:96])
    t3 = dot(e3[...], wv[96:128])
    t01 = dot(s_lo, t0) + dot(s_hi, t1)
    hc = lax.broadcasted_iota(jnp.int32, (1, 8), 1)
    oh2 = (hc == (h & 7)).astype(jnp.float32)
    oh3 = (hc == (h >> 3)).astype(jnp.float32)
    row23 = dot(oh2, t2) + dot(oh3, t3) + b[...]
    tt_ref[...] = t01 + row23


def _build_table(E0, E1, E2, E3, W, b2):
    full = lambda s: pl.BlockSpec(s, lambda h: tuple(0 for _ in s))
    return pl.pallas_call(
        _tables_body,
        grid=(NTT // 64,),
        in_specs=[
            full((BINS, ADIM)), full((BINS, ADIM)),
            full((BINS, ADIM)), full((BINS, ADIM)),
            full((4 * ADIM, NEMB)), full((1, NEMB)),
        ],
        out_specs=pl.BlockSpec((64, NEMB), lambda h: (h, 0)),
        out_shape=jax.ShapeDtypeStruct((NTT, NEMB), jnp.float32),
    )(E0, E1, E2, E3, W, b2)


_IDX_BLK = 8192


def _idx_body(a_ref, idx_ref):
    a = a_ref[...]
    idx_ref[...] = (a[:, 0:1] + 8 * a[:, 1:2]
                    + 64 * a[:, 2:3] + 512 * a[:, 3:4])


def _build_idx(a):
    return pl.pallas_call(
        _idx_body,
        grid=(ROWS // _IDX_BLK,),
        in_specs=[pl.BlockSpec((_IDX_BLK, 4), lambda i: (i, 0))],
        out_specs=pl.BlockSpec((_IDX_BLK, 1), lambda i: (i, 0)),
        out_shape=jax.ShapeDtypeStruct((ROWS, 1), jnp.int32),
    )(a)


def _gather_body(tt_hbm, idx_hbm, out_hbm, idx_v, buf0, buf1,
                 gsem, osem0, osem1):
    cid = lax.axis_index("c")
    sid = lax.axis_index("s")
    wid = sid * NC + cid
    base = wid * RPW

    pltpu.sync_copy(idx_hbm.at[pl.ds(base, RPW)], idx_v)

    def chunk(g, buf, osem, first):
        @pl.when(jnp.logical_not(first))
        def _():
            pltpu.make_async_copy(
                buf, out_hbm.at[pl.ds(base, CHUNK)], osem).wait()
        off = g * CHUNK
        pltpu.async_copy(
            tt_hbm.at[idx_v.at[pl.ds(off, CHUNK)]], buf, gsem).wait()
        pltpu.async_copy(buf, out_hbm.at[pl.ds(base + off, CHUNK)], osem)

    def body(i, carry):
        chunk(2 * i, buf0, osem0, i == 0)
        chunk(2 * i + 1, buf1, osem1, i == 0)
        return carry

    lax.fori_loop(0, NCHUNK // 2, body, 0)

    pltpu.make_async_copy(buf0, out_hbm.at[pl.ds(base, CHUNK)], osem0).wait()
    pltpu.make_async_copy(buf1, out_hbm.at[pl.ds(base, CHUNK)], osem1).wait()


@functools.partial(
    pl.kernel,
    out_type=jax.ShapeDtypeStruct((ROWS, NEMB), jnp.float32),
    mesh=plsc.VectorSubcoreMesh(core_axis_name="c", subcore_axis_name="s"),
    scratch_types=[
        pltpu.VMEM((RPW,), jnp.int32),
        pltpu.VMEM((CHUNK, NEMB), jnp.float32),
        pltpu.VMEM((CHUNK, NEMB), jnp.float32),
        pltpu.SemaphoreType.DMA,
        pltpu.SemaphoreType.DMA,
        pltpu.SemaphoreType.DMA,
    ],
)
def _gather_rows(tt_hbm, idx_hbm, out_hbm, *rest):
    _gather_body(tt_hbm, idx_hbm, out_hbm, *rest)


@jax.jit
def kernel(attributes, E0, E1, E2, E3, W, b):
    a = attributes.reshape(ROWS, 4).astype(jnp.int32)
    tt = _build_table(E0, E1, E2, E3, W, b.reshape(1, NEMB))
    idx = _build_idx(a)
    out = _gather_rows(tt, idx.reshape(ROWS))
    return out.reshape(B, BARS, NEMB)


# speedup vs baseline: 7.0555x; 1.1399x over previous
import functools

import jax
import jax.numpy as jnp
from jax import lax
from jax.experimental import pallas as pl
from jax.experimental.pallas import tpu as pltpu
from jax.experimental.pallas import tpu_sc as plsc

B = 16384
BARS = 8
BINS = 8
ADIM = 32
NEMB = 512
ROWS = B * BARS
NTT = BINS ** 4

NC = 2
NS = 16
NW = NC * NS
RPW = ROWS // NW
CHUNK = 64
NCHUNK = RPW // CHUNK


def _tables_body(e0, e1, e2, e3, w, b, tt_ref):
    r = lax.broadcasted_iota(jnp.int32, (64, 8), 0)
    c = lax.broadcasted_iota(jnp.int32, (64, 8), 1)
    s_lo = (c == (r & 7)).astype(jnp.float32)
    s_hi = (c == (r >> 3)).astype(jnp.float32)
    wv = w[...]
    dot = functools.partial(jnp.dot, preferred_element_type=jnp.float32)
    t0 = dot(e0[...], wv[0:32])
    t1 = dot(e1[...], wv[32:64])
    t2 = dot(e2[...], wv[64:96])
    t3 = dot(e3[...], wv[96:128])
    t01 = dot(s_lo, t0) + dot(s_hi, t1)
    t23 = dot(s_lo, t2) + dot(s_hi, t3) + b[...]
    rr = lax.broadcasted_iota(jnp.int32, (NTT, 64), 0)
    cc = lax.broadcasted_iota(jnp.int32, (NTT, 64), 1)
    g_lo = (cc == (rr & 63)).astype(jnp.float32)
    g_hi = (cc == (rr >> 6)).astype(jnp.float32)
    tt_ref[...] = dot(g_lo, t01) + dot(g_hi, t23)


def _build_table(E0, E1, E2, E3, W, b2):
    return pl.pallas_call(
        _tables_body,
        out_shape=jax.ShapeDtypeStruct((NTT, NEMB), jnp.float32),
    )(E0, E1, E2, E3, W, b2)


def _idx_body(a_ref, idx_ref):
    af = a_ref[...].astype(jnp.float32)
    r = lax.broadcasted_iota(jnp.int32, (512, 128), 0)
    c = lax.broadcasted_iota(jnp.int32, (512, 128), 1)
    w = (1 << (3 * (r & 3))).astype(jnp.float32)
    m = (c == (r >> 2)).astype(jnp.float32) * w
    idx_ref[...] = jnp.dot(
        af, m, preferred_element_type=jnp.float32).astype(jnp.int32)


def _build_idx(a2d):
    return pl.pallas_call(
        _idx_body,
        out_shape=jax.ShapeDtypeStruct((ROWS // 128, 128), jnp.int32),
    )(a2d)


def _gather_body(tt_hbm, idx_hbm, out_hbm, idx_v, buf0, buf1,
                 gsem, osem0, osem1):
    cid = lax.axis_index("c")
    sid = lax.axis_index("s")
    wid = sid * NC + cid
    base = wid * RPW

    pltpu.sync_copy(idx_hbm.at[pl.ds(base, RPW)], idx_v)

    def chunk(g, buf, osem, first):
        @pl.when(jnp.logical_not(first))
        def _():
            pltpu.make_async_copy(
                buf, out_hbm.at[pl.ds(base, CHUNK)], osem).wait()
        off = g * CHUNK
        pltpu.async_copy(
            tt_hbm.at[idx_v.at[pl.ds(off, CHUNK)]], buf, gsem).wait()
        pltpu.async_copy(buf, out_hbm.at[pl.ds(base + off, CHUNK)], osem)

    def body(i, carry):
        chunk(2 * i, buf0, osem0, i == 0)
        chunk(2 * i + 1, buf1, osem1, i == 0)
        return carry

    lax.fori_loop(0, NCHUNK // 2, body, 0)

    pltpu.make_async_copy(buf0, out_hbm.at[pl.ds(base, CHUNK)], osem0).wait()
    pltpu.make_async_copy(buf1, out_hbm.at[pl.ds(base, CHUNK)], osem1).wait()


@functools.partial(
    pl.kernel,
    out_type=jax.ShapeDtypeStruct((ROWS, NEMB), jnp.float32),
    mesh=plsc.VectorSubcoreMesh(core_axis_name="c", subcore_axis_name="s"),
    scratch_types=[
        pltpu.VMEM((RPW,), jnp.int32),
        pltpu.VMEM((CHUNK, NEMB), jnp.float32),
        pltpu.VMEM((CHUNK, NEMB), jnp.float32),
        pltpu.SemaphoreType.DMA,
        pltpu.SemaphoreType.DMA,
        pltpu.SemaphoreType.DMA,
    ],
)
def _gather_rows(tt_hbm, idx_hbm, out_hbm, *rest):
    _gather_body(tt_hbm, idx_hbm, out_hbm, *rest)


@jax.jit
def kernel(attributes, E0, E1, E2, E3, W, b):
    a2d = attributes.astype(jnp.int32).reshape(ROWS // 128, 512)
    tt = _build_table(E0, E1, E2, E3, W, b.reshape(1, NEMB))
    idx = _build_idx(a2d)
    out = _gather_rows(tt, idx.reshape(ROWS))
    return out.reshape(B, BARS, NEMB)


# speedup vs baseline: 9.5150x; 1.3486x over previous
import functools

import jax
import jax.numpy as jnp
from jax import lax
from jax.experimental import pallas as pl
from jax.experimental.pallas import tpu as pltpu
from jax.experimental.pallas import tpu_sc as plsc

B = 16384
BARS = 8
BINS = 8
ADIM = 32
NEMB = 512
ROWS = B * BARS
NTT = BINS ** 4

NC = 2
NS = 16
NW = NC * NS
RPW = ROWS // NW
CHUNK = 64
NCHUNK = RPW // CHUNK


def _tables_body(e0, e1, e2, e3, w, b, tt_ref):
    r = lax.broadcasted_iota(jnp.int32, (64, 8), 0)
    c = lax.broadcasted_iota(jnp.int32, (64, 8), 1)
    s_lo = (c == (r & 7)).astype(jnp.float32)
    s_hi = (c == (r >> 3)).astype(jnp.float32)
    wv = w[...]
    dot = functools.partial(jnp.dot, preferred_element_type=jnp.float32)
    t0 = dot(e0[...], wv[0:32])
    t1 = dot(e1[...], wv[32:64])
    t2 = dot(e2[...], wv[64:96])
    t3 = dot(e3[...], wv[96:128])
    t01 = dot(s_lo, t0) + dot(s_hi, t1)
    t23 = dot(s_lo, t2) + dot(s_hi, t3) + b[...]
    rr = lax.broadcasted_iota(jnp.int32, (NTT, 64), 0)
    cc = lax.broadcasted_iota(jnp.int32, (NTT, 64), 1)
    g_lo = (cc == (rr & 63)).astype(jnp.float32)
    g_hi = (cc == (rr >> 6)).astype(jnp.float32)
    tt_ref[...] = dot(g_lo, t01) + dot(g_hi, t23)


def _build_table(E0, E1, E2, E3, W, b2):
    return pl.pallas_call(
        _tables_body,
        out_shape=jax.ShapeDtypeStruct((NTT, NEMB), jnp.float32),
    )(E0, E1, E2, E3, W, b2)


def _idx_body(a_ref, idx_ref):
    idx_ref[...] = (a_ref[:, 0, :] + 8 * a_ref[:, 1, :]
                    + 64 * a_ref[:, 2, :] + 512 * a_ref[:, 3, :])


def _build_idx(a_t):
    return pl.pallas_call(
        _idx_body,
        out_shape=jax.ShapeDtypeStruct((BARS, B), jnp.int32),
    )(a_t)


BPW = B // (NW // BARS)


def _gather_body(tt_hbm, idxt_hbm, out_hbm, idx_v, buf0, buf1,
                 gsem, osem0, osem1):
    cid = lax.axis_index("c")
    sid = lax.axis_index("s")
    wid = sid * NC + cid
    bar = wid // (NW // BARS)
    bslot = wid % (NW // BARS)
    b0w = bslot * BPW

    pltpu.sync_copy(idxt_hbm.at[bar, pl.ds(b0w, BPW)], idx_v)

    def chunk(g, buf, osem, first):
        @pl.when(jnp.logical_not(first))
        def _():
            pltpu.make_async_copy(
                buf, out_hbm.at[pl.ds(b0w, CHUNK), bar], osem).wait()
        off = g * CHUNK
        pltpu.async_copy(
            tt_hbm.at[idx_v.at[pl.ds(off, CHUNK)]], buf, gsem).wait()
        pltpu.async_copy(
            buf, out_hbm.at[pl.ds(b0w + off, CHUNK), bar], osem)

    def body(i, carry):
        chunk(2 * i, buf0, osem0, i == 0)
        chunk(2 * i + 1, buf1, osem1, i == 0)
        return carry

    lax.fori_loop(0, NCHUNK // 2, body, 0)

    pltpu.make_async_copy(buf0, out_hbm.at[pl.ds(b0w, CHUNK), bar], osem0).wait()
    pltpu.make_async_copy(buf1, out_hbm.at[pl.ds(b0w, CHUNK), bar], osem1).wait()


@functools.partial(
    pl.kernel,
    out_type=jax.ShapeDtypeStruct((B, BARS, NEMB), jnp.float32),
    mesh=plsc.VectorSubcoreMesh(core_axis_name="c", subcore_axis_name="s"),
    scratch_types=[
        pltpu.VMEM((BPW,), jnp.int32),
        pltpu.VMEM((CHUNK, NEMB), jnp.float32),
        pltpu.VMEM((CHUNK, NEMB), jnp.float32),
        pltpu.SemaphoreType.DMA,
        pltpu.SemaphoreType.DMA,
        pltpu.SemaphoreType.DMA,
    ],
)
def _gather_rows(tt_hbm, idxt_hbm, out_hbm, *rest):
    _gather_body(tt_hbm, idxt_hbm, out_hbm, *rest)


@jax.jit
def kernel(attributes, E0, E1, E2, E3, W, b):
    a_t = attributes.astype(jnp.int32).transpose(1, 2, 0)
    tt = _build_table(E0, E1, E2, E3, W, b.reshape(1, NEMB))
    idxt = _build_idx(a_t)
    return _gather_rows(tt, idxt)


# speedup vs baseline: 10.4593x; 1.0992x over previous
import functools

import jax
import jax.numpy as jnp
from jax import lax
from jax.experimental import pallas as pl
from jax.experimental.pallas import tpu as pltpu
from jax.experimental.pallas import tpu_sc as plsc

B = 16384
BARS = 8
BINS = 8
ADIM = 32
NEMB = 512
ROWS = B * BARS
NTT = BINS ** 4

NC = 2
NS = 16
NW = NC * NS
RPW = ROWS // NW
CHUNK = 64
NCHUNK = RPW // CHUNK


def _tables_body(e0, e1, e2, e3, w, b, tt_ref):
    r = lax.broadcasted_iota(jnp.int32, (64, 8), 0)
    c = lax.broadcasted_iota(jnp.int32, (64, 8), 1)
    s_lo = (c == (r & 7)).astype(jnp.float32)
    s_hi = (c == (r >> 3)).astype(jnp.float32)
    wv = w[...]
    dot = functools.partial(jnp.dot, preferred_element_type=jnp.float32)
    t0 = dot(e0[...], wv[0:32])
    t1 = dot(e1[...], wv[32:64])
    t2 = dot(e2[...], wv[64:96])
    t3 = dot(e3[...], wv[96:128])
    t01 = dot(s_lo, t0) + dot(s_hi, t1)
    t23 = dot(s_lo, t2) + dot(s_hi, t3) + b[...]
    rr = lax.broadcasted_iota(jnp.int32, (NTT, 64), 0)
    cc = lax.broadcasted_iota(jnp.int32, (NTT, 64), 1)
    g_lo = (cc == (rr & 63)).astype(jnp.float32)
    g_hi = (cc == (rr >> 6)).astype(jnp.float32)
    tt_ref[...] = dot(g_lo, t01) + dot(g_hi, t23)


def _build_table(E0, E1, E2, E3, W, b2):
    return pl.pallas_call(
        _tables_body,
        out_shape=jax.ShapeDtypeStruct((NTT, NEMB), jnp.float32),
    )(E0, E1, E2, E3, W, b2)


def _idx_body(a_ref, idx_ref):
    idx_ref[...] = (a_ref[:, 0, :] + 8 * a_ref[:, 1, :]
                    + 64 * a_ref[:, 2, :] + 512 * a_ref[:, 3, :])


def _build_idx(a_t):
    return pl.pallas_call(
        _idx_body,
        out_shape=jax.ShapeDtypeStruct((BARS, B), jnp.int32),
    )(a_t)


BPW = B // (NW // BARS)


def _gather_body(tt_hbm, idxt_hbm, out_hbm, idx_v, buf0, buf1,
                 gsem0, gsem1, osem0, osem1):
    cid = lax.axis_index("c")
    sid = lax.axis_index("s")
    wid = sid * NC + cid
    bar = wid // (NW // BARS)
    bslot = wid % (NW // BARS)
    b0w = bslot * BPW

    pltpu.sync_copy(idxt_hbm.at[bar, pl.ds(b0w, BPW)], idx_v)

    def gather(g, buf, gs):
        return pltpu.make_async_copy(
            tt_hbm.at[idx_v.at[pl.ds(g * CHUNK, CHUNK)]], buf, gs)

    def writeout(g, buf, os):
        return pltpu.make_async_copy(
            buf, out_hbm.at[pl.ds(b0w + g * CHUNK, CHUNK), bar], os)

    gather(0, buf0, gsem0).start()
    gather(1, buf1, gsem1).start()

    def slot(g, buf, gs, os):
        gather(g, buf, gs).wait()
        writeout(g, buf, os).start()

        @pl.when(g + 2 < NCHUNK)
        def _():
            writeout(g, buf, os).wait()
            gather(g + 2, buf, gs).start()

    def body(i, carry):
        slot(2 * i, buf0, gsem0, osem0)
        slot(2 * i + 1, buf1, gsem1, osem1)
        return carry

    lax.fori_loop(0, NCHUNK // 2, body, 0)

    writeout(NCHUNK - 2, buf0, osem0).wait()
    writeout(NCHUNK - 1, buf1, osem1).wait()


@functools.partial(
    pl.kernel,
    out_type=jax.ShapeDtypeStruct((B, BARS, NEMB), jnp.float32),
    mesh=plsc.VectorSubcoreMesh(core_axis_name="c", subcore_axis_name="s"),
    scratch_types=[
        pltpu.VMEM((BPW,), jnp.int32),
        pltpu.VMEM((CHUNK, NEMB), jnp.float32),
        pltpu.VMEM((CHUNK, NEMB), jnp.float32),
        pltpu.SemaphoreType.DMA,
        pltpu.SemaphoreType.DMA,
        pltpu.SemaphoreType.DMA,
        pltpu.SemaphoreType.DMA,
    ],
)
def _gather_rows(tt_hbm, idxt_hbm, out_hbm, *rest):
    _gather_body(tt_hbm, idxt_hbm, out_hbm, *rest)


@jax.jit
def kernel(attributes, E0, E1, E2, E3, W, b):
    a_t = attributes.astype(jnp.int32).transpose(1, 2, 0)
    tt = _build_table(E0, E1, E2, E3, W, b.reshape(1, NEMB))
    idxt = _build_idx(a_t)
    return _gather_rows(tt, idxt)


# speedup vs baseline: 10.4958x; 1.0035x over previous
import functools

import jax
import jax.numpy as jnp
from jax import lax
from jax.experimental import pallas as pl
from jax.experimental.pallas import tpu as pltpu
from jax.experimental.pallas import tpu_sc as plsc

B = 16384
BARS = 8
BINS = 8
ADIM = 32
NEMB = 512
ROWS = B * BARS
NTT = BINS ** 4

NC = 2
NS = 16
NW = NC * NS
RPW = ROWS // NW
CHUNK = 32
NBUF = 4
NCHUNK = RPW // CHUNK


def _tables_body(e0, e1, e2, e3, w, b, tt_ref):
    r = lax.broadcasted_iota(jnp.int32, (64, 8), 0)
    c = lax.broadcasted_iota(jnp.int32, (64, 8), 1)
    s_lo = (c == (r & 7)).astype(jnp.float32)
    s_hi = (c == (r >> 3)).astype(jnp.float32)
    wv = w[...]
    dot = functools.partial(jnp.dot, preferred_element_type=jnp.float32)
    t0 = dot(e0[...], wv[0:32])
    t1 = dot(e1[...], wv[32:64])
    t2 = dot(e2[...], wv[64:96])
    t3 = dot(e3[...], wv[96:128])
    t01 = dot(s_lo, t0) + dot(s_hi, t1)
    t23 = dot(s_lo, t2) + dot(s_hi, t3) + b[...]
    rr = lax.broadcasted_iota(jnp.int32, (NTT, 64), 0)
    cc = lax.broadcasted_iota(jnp.int32, (NTT, 64), 1)
    g_lo = (cc == (rr & 63)).astype(jnp.float32)
    g_hi = (cc == (rr >> 6)).astype(jnp.float32)
    tt_ref[...] = dot(g_lo, t01) + dot(g_hi, t23)


def _build_table(E0, E1, E2, E3, W, b2):
    return pl.pallas_call(
        _tables_body,
        out_shape=jax.ShapeDtypeStruct((NTT, NEMB), jnp.float32),
    )(E0, E1, E2, E3, W, b2)


def _idx_body(a_ref, idx_ref):
    idx_ref[...] = (a_ref[:, 0, :] + 8 * a_ref[:, 1, :]
                    + 64 * a_ref[:, 2, :] + 512 * a_ref[:, 3, :])


def _build_idx(a_t):
    return pl.pallas_call(
        _idx_body,
        out_shape=jax.ShapeDtypeStruct((BARS, B), jnp.int32),
    )(a_t)


BPW = B // (NW // BARS)


def _gather_body(tt_hbm, idxt_hbm, out_hbm, idx_v, *rest):
    bufs = rest[:NBUF]
    gsems = rest[NBUF:2 * NBUF]
    osems = rest[2 * NBUF:3 * NBUF]
    cid = lax.axis_index("c")
    sid = lax.axis_index("s")
    wid = sid * NC + cid
    bar = wid // (NW // BARS)
    bslot = wid % (NW // BARS)
    b0w = bslot * BPW

    pltpu.sync_copy(idxt_hbm.at[bar, pl.ds(b0w, BPW)], idx_v)

    def gather(g, buf, gs):
        return pltpu.make_async_copy(
            tt_hbm.at[idx_v.at[pl.ds(g * CHUNK, CHUNK)]], buf, gs)

    def writeout(g, buf, os):
        return pltpu.make_async_copy(
            buf, out_hbm.at[pl.ds(b0w + g * CHUNK, CHUNK), bar], os)

    for j in range(NBUF):
        gather(j, bufs[j], gsems[j]).start()

    def slot(g, buf, gs, os):
        gather(g, buf, gs).wait()
        writeout(g, buf, os).start()

        @pl.when(g + NBUF < NCHUNK)
        def _():
            writeout(g, buf, os).wait()
            gather(g + NBUF, buf, gs).start()

    def body(i, carry):
        for j in range(NBUF):
            slot(NBUF * i + j, bufs[j], gsems[j], osems[j])
        return carry

    lax.fori_loop(0, NCHUNK // NBUF, body, 0)

    for j in range(NBUF):
        writeout(NCHUNK - NBUF + j, bufs[j], osems[j]).wait()


@functools.partial(
    pl.kernel,
    out_type=jax.ShapeDtypeStruct((B, BARS, NEMB), jnp.float32),
    mesh=plsc.VectorSubcoreMesh(core_axis_name="c", subcore_axis_name="s"),
    scratch_types=(
        [pltpu.VMEM((BPW,), jnp.int32)]
        + [pltpu.VMEM((CHUNK, NEMB), jnp.float32)] * NBUF
        + [pltpu.SemaphoreType.DMA] * (2 * NBUF)
    ),
)
def _gather_rows(tt_hbm, idxt_hbm, out_hbm, *rest):
    _gather_body(tt_hbm, idxt_hbm, out_hbm, *rest)


@jax.jit
def kernel(attributes, E0, E1, E2, E3, W, b):
    a_t = attributes.astype(jnp.int32).transpose(1, 2, 0)
    tt = _build_table(E0, E1, E2, E3, W, b.reshape(1, NEMB))
    idxt = _build_idx(a_t)
    return _gather_rows(tt, idxt)
